# baseline (speedup 1.0000x reference)
import jax
import jax.numpy as jnp
from jax import lax
from jax.experimental import pallas as pl
from jax.experimental.pallas import tpu as pltpu
from jax.experimental.pallas import tpu_sc as plsc

_V = 1000
_C = 128
_B = 1024
_L = 200
_LANES = 16
_NB = 13

_NW = 32
_BPW = _B // _NW
_NS = 4

_CH0 = 104
_CH1 = _L - _CH0


def _sc_body(x_hbm, mask_hbm, tab_hbm, out_hbm,
             idx_all, mask_all, r0, r1, r2, r3, tab_sh,
             g0, g1, g2, g3, o0, o1, o2, o3):
    sid = lax.axis_index("s")
    wid = sid * 2 + lax.axis_index("c")
    iota = lax.broadcasted_iota(jnp.int32, (_LANES,), 0)
    perms = [(iota + s) & 15 for s in range(_LANES)]
    b0 = wid * _BPW
    rows = (r0, r1, r2, r3)
    gsems = (g0, g1, g2, g3)
    osems = (o0, o1, o2, o3)

    def start_gather(i, rows_v, sem):
        pltpu.async_copy(tab_sh.at[idx_all.at[i, pl.ds(0, _CH0)]],
                         rows_v.at[pl.ds(0, _CH0)], sem)
        pltpu.async_copy(tab_sh.at[idx_all.at[i, pl.ds(_CH0, _CH1)]],
                         rows_v.at[pl.ds(_CH0, _CH1)], sem)

    def wait_gather(i, rows_v, sem):
        pltpu.make_async_copy(tab_sh.at[idx_all.at[i, pl.ds(0, _CH0)]],
                              rows_v.at[pl.ds(0, _CH0)], sem).wait()
        pltpu.make_async_copy(tab_sh.at[idx_all.at[i, pl.ds(_CH0, _CH1)]],
                              rows_v.at[pl.ds(_CH0, _CH1)], sem).wait()

    pltpu.sync_copy(x_hbm.at[pl.ds(b0, _BPW)], idx_all)

    @pl.when(sid == 0)
    def _():
        pltpu.sync_copy(tab_hbm, tab_sh)

    pltpu.sync_copy(mask_hbm.at[pl.ds(b0, _BPW)], mask_all)
    plsc.subcore_barrier()

    start_gather(0, rows[0], gsems[0])
    start_gather(1, rows[1], gsems[1])

    def per_quad(q, carry):
        for par in range(_NS):
            i = _NS * q + par
            rows_cur = rows[par]
            nxt = (par + 2) % _NS

            @pl.when(i >= 2)
            def _():
                pltpu.make_async_copy(rows[nxt], out_hbm.at[b0 + i - 2],
                                      osems[nxt]).wait()

            @pl.when(i + 2 < _BPW)
            def _():
                start_gather(i + 2, rows[nxt], gsems[nxt])

            wait_gather(i, rows_cur, gsems[par])

            ivec = jnp.full((_LANES,), i, jnp.int32)

            def per_lb(lb, cc, rows_cur=rows_cur, ivec=ivec):
                l0 = lb * 16
                lvec = jnp.minimum(iota + l0, _L - 1)
                valid = iota < (_L - l0)
                m = plsc.load_gather(mask_all, [ivec, lvec])

                @plsc.parallel_loop(0, _C // 16)
                def _ct(ct, lvec=lvec, m=m, valid=valid, rows_cur=rows_cur):
                    c0 = ct * 16
                    for s in range(_LANES):
                        cvec = perms[s] + c0
                        vals = plsc.load_gather(rows_cur, [lvec, cvec]) * m
                        plsc.store_scatter(rows_cur, [lvec, cvec], vals,
                                           mask=valid)

                return cc

            lax.fori_loop(0, _NB, per_lb, 0)

            pltpu.async_copy(rows_cur, out_hbm.at[b0 + i], osems[par])
        return carry

    lax.fori_loop(0, _BPW // _NS, per_quad, 0)

    for par in (2, 3):
        pltpu.make_async_copy(rows[par], out_hbm.at[b0 + _BPW - 4 + par],
                              osems[par]).wait()


def kernel(x, mask, emb_weight):
    x32 = x.astype(jnp.int32)
    mask2 = mask.reshape(_B, _L)
    mesh = plsc.VectorSubcoreMesh(core_axis_name="c", subcore_axis_name="s")
    run = pl.kernel(
        _sc_body,
        out_type=jax.ShapeDtypeStruct((_B, _L, _C), jnp.float32),
        mesh=mesh,
        compiler_params=pltpu.CompilerParams(
            needs_layout_passes=False, use_tc_tiling_on_sc=False),
        scratch_types=[
            pltpu.VMEM((_BPW, _L), jnp.int32),
            pltpu.VMEM((_BPW, _L), jnp.float32),
            pltpu.VMEM((_L, _C), jnp.float32),
            pltpu.VMEM((_L, _C), jnp.float32),
            pltpu.VMEM((_L, _C), jnp.float32),
            pltpu.VMEM((_L, _C), jnp.float32),
            pltpu.VMEM_SHARED((_V, _C), jnp.float32),
            pltpu.SemaphoreType.DMA,
            pltpu.SemaphoreType.DMA,
            pltpu.SemaphoreType.DMA,
            pltpu.SemaphoreType.DMA,
            pltpu.SemaphoreType.DMA,
            pltpu.SemaphoreType.DMA,
            pltpu.SemaphoreType.DMA,
            pltpu.SemaphoreType.DMA,
        ],
    )
    y = run(x32, mask2, emb_weight)
    return jnp.swapaxes(y, 1, 2)
